# Initial kernel scaffold; baseline (speedup 1.0000x reference)
import jax
import jax.numpy as jnp
from jax.experimental import pallas as pl


def kernel(x, edge_index, W1a, b1a, W1b, b1b, W2a, b2a, W2b, b2b, Wf, bf):
    raise NotImplementedError("write your pallas kernel here")


# speedup vs baseline: 5.3480x; 5.3480x over previous
import functools

import jax
import jax.numpy as jnp
from jax import lax
from jax.experimental import pallas as pl
from jax.experimental.pallas import tpu as pltpu
from jax.experimental.pallas import tpu_sc as plsc

N = 10000
E = 320000
D = 128
H = 32

NC = 2
NS = 16
NW = NC * NS
EPW = E // NW
CH = 128
NFULL = EPW // CH
TAIL = EPW - NFULL * CH
RPT = 624
RPT_LAST = N - 15 * RPT


def _make_sc_scatter(W, P):
    mesh = plsc.VectorSubcoreMesh(core_axis_name="c", subcore_axis_name="s")

    @functools.partial(
        pl.kernel,
        out_type=jax.ShapeDtypeStruct((P * 2 * N, W), jnp.float32),
        mesh=mesh,
        scratch_types=[
            pltpu.VMEM((CH,), jnp.int32),
            pltpu.VMEM((CH,), jnp.int32),
            pltpu.VMEM((CH, W), jnp.float32),
            pltpu.VMEM((TAIL,), jnp.int32),
            pltpu.VMEM((TAIL,), jnp.int32),
            pltpu.VMEM((TAIL, W), jnp.float32),
            pltpu.VMEM((RPT_LAST, W), jnp.float32),
            pltpu.VMEM_SHARED((N, W), jnp.float32),
            pltpu.SemaphoreType.DMA,
        ],
        compiler_params=pltpu.CompilerParams(use_tc_tiling_on_sc=False),
    )
    def k(*refs):
        y_hbm = refs[0]
        srcs = refs[1:1 + P]
        dst_hbm = refs[1 + P]
        zeros_hbm = refs[2 + P]
        out_hbm = refs[3 + P]
        (srcv, dstv, rows, srcv_t, dstv_t, rows_t, stage, accum,
         sem) = refs[4 + P:]

        c = lax.axis_index("c")
        s = lax.axis_index("s")
        row0 = pl.multiple_of(s * RPT, 8)
        base = pl.multiple_of((c * NS + s) * EPW, 8)

        for p in range(P):
            src_hbm = srcs[p]

            @pl.when(s < NS - 1)
            def _():
                pltpu.sync_copy(zeros_hbm.at[pl.ds(row0, RPT)],
                                stage.at[pl.ds(0, RPT)])
                pltpu.sync_copy(stage.at[pl.ds(0, RPT)],
                                accum.at[pl.ds(row0, RPT)])

            @pl.when(s == NS - 1)
            def _():
                pltpu.sync_copy(zeros_hbm.at[pl.ds(row0, RPT_LAST)], stage)
                pltpu.sync_copy(stage, accum.at[pl.ds(row0, RPT_LAST)])

            plsc.subcore_barrier()

            def body(j, carry):
                off = pl.multiple_of(base + j * CH, 8)
                pltpu.sync_copy(src_hbm.at[pl.ds(off, CH)], srcv)
                pltpu.sync_copy(dst_hbm.at[pl.ds(off, CH)], dstv)
                pltpu.async_copy(y_hbm.at[srcv], rows, sem).wait()
                pltpu.sync_copy(rows, accum.at[dstv], add=True)
                return carry

            lax.fori_loop(0, NFULL, body, 0)

            off = pl.multiple_of(base + NFULL * CH, 8)
            pltpu.sync_copy(src_hbm.at[pl.ds(off, TAIL)], srcv_t)
            pltpu.sync_copy(dst_hbm.at[pl.ds(off, TAIL)], dstv_t)
            pltpu.async_copy(y_hbm.at[srcv_t], rows_t, sem).wait()
            pltpu.sync_copy(rows_t, accum.at[dstv_t], add=True)

            plsc.subcore_barrier()

            out0 = pl.multiple_of((p * 2 + c) * N + s * RPT, 8)

            @pl.when(s < NS - 1)
            def _():
                pltpu.sync_copy(accum.at[pl.ds(row0, RPT)],
                                stage.at[pl.ds(0, RPT)])
                pltpu.sync_copy(stage.at[pl.ds(0, RPT)],
                                out_hbm.at[pl.ds(out0, RPT)])

            @pl.when(s == NS - 1)
            def _():
                pltpu.sync_copy(accum.at[pl.ds(row0, RPT_LAST)], stage)
                pltpu.sync_copy(stage, out_hbm.at[pl.ds(out0, RPT_LAST)])

    return k


_sc_scatter_d = _make_sc_scatter(D // 2, 2)
_sc_scatter_h = _make_sc_scatter(H, 1)


def _tc_conv1(x, pL0, pL1, pR0, pR1, W1a, b1a, W1b, b1b):
    def body(x_ref, pL0_ref, pL1_ref, pR0_ref, pR1_ref, wa_ref, ba_ref,
             wb_ref, bb_ref, o_ref):
        agg = jnp.concatenate([pL0_ref[...] + pL1_ref[...],
                               pR0_ref[...] + pR1_ref[...]], axis=1)
        h = x_ref[...] + agg
        u = jnp.maximum(jnp.dot(h, wa_ref[...],
                                preferred_element_type=jnp.float32)
                        + ba_ref[...], 0.0)
        v = jnp.dot(u, wb_ref[...], preferred_element_type=jnp.float32)
        o_ref[...] = jnp.maximum(v + bb_ref[...], 0.0)
    return pl.pallas_call(
        body, out_shape=jax.ShapeDtypeStruct((N, H), jnp.float32))(
            x, pL0, pL1, pR0, pR1, W1a, b1a.reshape(1, H), W1b,
            b1b.reshape(1, H))


def _tc_conv2(h1, q0, q1, W2a, b2a, W2b, b2b, Wf, bf):
    def body(h_ref, q0_ref, q1_ref, wa_ref, ba_ref, wb_ref, bb_ref,
             wf_ref, bf_ref, o_ref):
        g = h_ref[...] + (q0_ref[...] + q1_ref[...])
        u = jnp.maximum(jnp.dot(g, wa_ref[...],
                                preferred_element_type=jnp.float32)
                        + ba_ref[...], 0.0)
        v = jnp.dot(u, wb_ref[...], preferred_element_type=jnp.float32)
        h2 = jnp.maximum(v + bb_ref[...], 0.0)
        o_ref[...] = jnp.dot(h2, wf_ref[...],
                             preferred_element_type=jnp.float32) + bf_ref[...]
    return pl.pallas_call(
        body, out_shape=jax.ShapeDtypeStruct((N, 1), jnp.float32))(
            h1, q0, q1, W2a, b2a.reshape(1, H), W2b, b2b.reshape(1, H),
            Wf, bf.reshape(1, 1))


def kernel(x, edge_index, W1a, b1a, W1b, b1b, W2a, b2a, W2b, b2b, Wf, bf):
    src = edge_index[0]
    dst = edge_index[1]
    x2 = x.reshape(2 * N, D // 2)
    src_even = src * 2
    src_odd = src * 2 + 1
    zeros_d = jnp.zeros((N, D // 2), jnp.float32)
    zeros_h = jnp.zeros((N, H), jnp.float32)

    parts1 = _sc_scatter_d(x2, src_even, src_odd, dst, zeros_d)
    h1 = _tc_conv1(x, parts1[:N], parts1[N:2 * N],
                   parts1[2 * N:3 * N], parts1[3 * N:],
                   W1a, b1a, W1b, b1b)
    parts2 = _sc_scatter_h(h1, src, dst, zeros_h)
    return _tc_conv2(h1, parts2[:N], parts2[N:], W2a, b2a, W2b, b2b, Wf, bf)


# speedup vs baseline: 9.5628x; 1.7881x over previous
import functools

import jax
import jax.numpy as jnp
from jax import lax
from jax.experimental import pallas as pl
from jax.experimental.pallas import tpu as pltpu
from jax.experimental.pallas import tpu_sc as plsc

N = 10000
E = 320000
D = 128
H = 32

NC = 2
NS = 16
NW = NC * NS
EPW = E // NW
CH = 128
NFULL = EPW // CH
TAIL = EPW - NFULL * CH
RPT = 624
RPT_LAST = N - 15 * RPT


def _make_sc_scatter(W, P):
    mesh = plsc.VectorSubcoreMesh(core_axis_name="c", subcore_axis_name="s")

    @functools.partial(
        pl.kernel,
        out_type=jax.ShapeDtypeStruct((P * 2 * N, W), jnp.float32),
        mesh=mesh,
        scratch_types=[
            pltpu.VMEM((CH,), jnp.int32),
            pltpu.VMEM((CH,), jnp.int32),
            pltpu.VMEM((CH,), jnp.int32),
            pltpu.VMEM((CH,), jnp.int32),
            pltpu.VMEM((CH, W), jnp.float32),
            pltpu.VMEM((CH, W), jnp.float32),
            pltpu.VMEM((TAIL,), jnp.int32),
            pltpu.VMEM((TAIL,), jnp.int32),
            pltpu.VMEM((TAIL, W), jnp.float32),
            pltpu.VMEM((RPT_LAST, W), jnp.float32),
            pltpu.VMEM_SHARED((N, W), jnp.float32),
            pltpu.SemaphoreType.DMA,
            pltpu.SemaphoreType.DMA,
            pltpu.SemaphoreType.DMA,
            pltpu.SemaphoreType.DMA,
        ],
        compiler_params=pltpu.CompilerParams(use_tc_tiling_on_sc=False),
    )
    def k(*refs):
        y_hbm = refs[0]
        srcs = refs[1:1 + P]
        dst_hbm = refs[1 + P]
        zeros_hbm = refs[2 + P]
        out_hbm = refs[3 + P]
        (srcv0, srcv1, dstv0, dstv1, rows0, rows1, srcv_t, dstv_t, rows_t,
         stage, accum, semi0, semi1, semg0, semg1) = refs[4 + P:]
        srcv = (srcv0, srcv1)
        dstv = (dstv0, dstv1)
        rows = (rows0, rows1)
        semi = (semi0, semi1)
        semg = (semg0, semg1)

        c = lax.axis_index("c")
        s = lax.axis_index("s")
        row0 = pl.multiple_of(s * RPT, 8)
        base = pl.multiple_of((c * NS + s) * EPW, 8)

        for p in range(P):
            src_hbm = srcs[p]

            @pl.when(s < NS - 1)
            def _():
                pltpu.sync_copy(zeros_hbm.at[pl.ds(row0, RPT)],
                                stage.at[pl.ds(0, RPT)])
                pltpu.sync_copy(stage.at[pl.ds(0, RPT)],
                                accum.at[pl.ds(row0, RPT)])

            @pl.when(s == NS - 1)
            def _():
                pltpu.sync_copy(zeros_hbm.at[pl.ds(row0, RPT_LAST)], stage)
                pltpu.sync_copy(stage, accum.at[pl.ds(row0, RPT_LAST)])

            plsc.subcore_barrier()

            def issue_idx(j, b):
                off = pl.multiple_of(base + j * CH, 8)
                pltpu.async_copy(src_hbm.at[pl.ds(off, CH)], srcv[b], semi[b])
                pltpu.async_copy(dst_hbm.at[pl.ds(off, CH)], dstv[b], semi[b])

            def wait_idx(b):
                pltpu.make_async_copy(
                    src_hbm.at[pl.ds(0, CH)], srcv[b], semi[b]).wait()
                pltpu.make_async_copy(
                    dst_hbm.at[pl.ds(0, CH)], dstv[b], semi[b]).wait()

            def issue_gather(b):
                pltpu.async_copy(y_hbm.at[srcv[b]], rows[b], semg[b])

            def wait_gather(b):
                pltpu.make_async_copy(y_hbm.at[srcv[b]], rows[b],
                                      semg[b]).wait()

            issue_idx(0, 0)
            issue_idx(1, 1)
            wait_idx(0)
            issue_gather(0)

            def body(jj, carry):
                for b in (0, 1):
                    j = 2 * jj + b
                    wait_gather(b)

                    @pl.when(j + 1 < NFULL)
                    def _():
                        wait_idx(1 - b)
                        issue_gather(1 - b)

                    pltpu.sync_copy(rows[b], accum.at[dstv[b]], add=True)

                    @pl.when(j + 2 < NFULL)
                    def _():
                        issue_idx(j + 2, b)
                return carry

            lax.fori_loop(0, NFULL // 2, body, 0)

            off = pl.multiple_of(base + NFULL * CH, 8)
            pltpu.sync_copy(src_hbm.at[pl.ds(off, TAIL)], srcv_t)
            pltpu.sync_copy(dst_hbm.at[pl.ds(off, TAIL)], dstv_t)
            pltpu.async_copy(y_hbm.at[srcv_t], rows_t, semg0).wait()
            pltpu.sync_copy(rows_t, accum.at[dstv_t], add=True)

            plsc.subcore_barrier()

            out0 = pl.multiple_of((p * 2 + c) * N + s * RPT, 8)

            @pl.when(s < NS - 1)
            def _():
                pltpu.sync_copy(accum.at[pl.ds(row0, RPT)],
                                stage.at[pl.ds(0, RPT)])
                pltpu.sync_copy(stage.at[pl.ds(0, RPT)],
                                out_hbm.at[pl.ds(out0, RPT)])

            @pl.when(s == NS - 1)
            def _():
                pltpu.sync_copy(accum.at[pl.ds(row0, RPT_LAST)], stage)
                pltpu.sync_copy(stage, out_hbm.at[pl.ds(out0, RPT_LAST)])

    return k


_sc_scatter_d = _make_sc_scatter(D // 2, 2)
_sc_scatter_h = _make_sc_scatter(H, 1)


def _tc_conv1(x, pL0, pL1, pR0, pR1, W1a, b1a, W1b, b1b):
    def body(x_ref, pL0_ref, pL1_ref, pR0_ref, pR1_ref, wa_ref, ba_ref,
             wb_ref, bb_ref, o_ref):
        agg = jnp.concatenate([pL0_ref[...] + pL1_ref[...],
                               pR0_ref[...] + pR1_ref[...]], axis=1)
        h = x_ref[...] + agg
        u = jnp.maximum(jnp.dot(h, wa_ref[...],
                                preferred_element_type=jnp.float32)
                        + ba_ref[...], 0.0)
        v = jnp.dot(u, wb_ref[...], preferred_element_type=jnp.float32)
        o_ref[...] = jnp.maximum(v + bb_ref[...], 0.0)
    return pl.pallas_call(
        body, out_shape=jax.ShapeDtypeStruct((N, H), jnp.float32))(
            x, pL0, pL1, pR0, pR1, W1a, b1a.reshape(1, H), W1b,
            b1b.reshape(1, H))


def _tc_conv2(h1, q0, q1, W2a, b2a, W2b, b2b, Wf, bf):
    def body(h_ref, q0_ref, q1_ref, wa_ref, ba_ref, wb_ref, bb_ref,
             wf_ref, bf_ref, o_ref):
        g = h_ref[...] + (q0_ref[...] + q1_ref[...])
        u = jnp.maximum(jnp.dot(g, wa_ref[...],
                                preferred_element_type=jnp.float32)
                        + ba_ref[...], 0.0)
        v = jnp.dot(u, wb_ref[...], preferred_element_type=jnp.float32)
        h2 = jnp.maximum(v + bb_ref[...], 0.0)
        o_ref[...] = jnp.dot(h2, wf_ref[...],
                             preferred_element_type=jnp.float32) + bf_ref[...]
    return pl.pallas_call(
        body, out_shape=jax.ShapeDtypeStruct((N, 1), jnp.float32))(
            h1, q0, q1, W2a, b2a.reshape(1, H), W2b, b2b.reshape(1, H),
            Wf, bf.reshape(1, 1))


def kernel(x, edge_index, W1a, b1a, W1b, b1b, W2a, b2a, W2b, b2b, Wf, bf):
    src = edge_index[0]
    dst = edge_index[1]
    x2 = x.reshape(2 * N, D // 2)
    src_even = src * 2
    src_odd = src * 2 + 1
    zeros_d = jnp.zeros((N, D // 2), jnp.float32)
    zeros_h = jnp.zeros((N, H), jnp.float32)

    parts1 = _sc_scatter_d(x2, src_even, src_odd, dst, zeros_d)
    h1 = _tc_conv1(x, parts1[:N], parts1[N:2 * N],
                   parts1[2 * N:3 * N], parts1[3 * N:],
                   W1a, b1a, W1b, b1b)
    parts2 = _sc_scatter_h(h1, src, dst, zeros_h)
    return _tc_conv2(h1, parts2[:N], parts2[N:], W2a, b2a, W2b, b2b, Wf, bf)


# speedup vs baseline: 11.0407x; 1.1546x over previous
import functools

import jax
import jax.numpy as jnp
from jax import lax
from jax.experimental import pallas as pl
from jax.experimental.pallas import tpu as pltpu
from jax.experimental.pallas import tpu_sc as plsc

N = 10000
E = 320000
D = 128
H = 32

NC = 2
NS = 16
CH = 128
RPT = 624
RPT_LAST = N - 15 * RPT

EPT1 = E // NS
NF1 = EPT1 // CH
TAIL1 = EPT1 - NF1 * CH

EPT2 = E // (NC * NS)
NF2 = EPT2 // CH
TAIL2 = EPT2 - NF2 * CH


def _zero_accum(zeros_hbm, accum, stage, s):
    row0 = pl.multiple_of(s * RPT, 8)

    @pl.when(s < NS - 1)
    def _():
        pltpu.sync_copy(zeros_hbm.at[pl.ds(row0, RPT)],
                        stage.at[pl.ds(0, RPT)])
        pltpu.sync_copy(stage.at[pl.ds(0, RPT)], accum.at[pl.ds(row0, RPT)])

    @pl.when(s == NS - 1)
    def _():
        pltpu.sync_copy(zeros_hbm.at[pl.ds(row0, RPT_LAST)], stage)
        pltpu.sync_copy(stage, accum.at[pl.ds(row0, RPT_LAST)])


def _readout(accum, out_hbm, stage, s, slab):
    row0 = pl.multiple_of(s * RPT, 8)
    out0 = pl.multiple_of(slab * N + s * RPT, 8)

    @pl.when(s < NS - 1)
    def _():
        pltpu.sync_copy(accum.at[pl.ds(row0, RPT)], stage.at[pl.ds(0, RPT)])
        pltpu.sync_copy(stage.at[pl.ds(0, RPT)],
                        out_hbm.at[pl.ds(out0, RPT)])

    @pl.when(s == NS - 1)
    def _():
        pltpu.sync_copy(accum.at[pl.ds(row0, RPT_LAST)], stage)
        pltpu.sync_copy(stage, out_hbm.at[pl.ds(out0, RPT_LAST)])


def _edge_loop(y_hbm, src_hbm, dst_hbm, accum, srcv, dstv, rows, semi, semg,
               src_base, dst_base, nf):
    def issue_idx(j, b):
        soff = pl.multiple_of(src_base + j * CH, 8)
        doff = pl.multiple_of(dst_base + j * CH, 8)
        pltpu.async_copy(src_hbm.at[pl.ds(soff, CH)], srcv[b], semi[b])
        pltpu.async_copy(dst_hbm.at[pl.ds(doff, CH)], dstv[b], semi[b])

    def wait_idx(b):
        pltpu.make_async_copy(src_hbm.at[pl.ds(0, CH)], srcv[b],
                              semi[b]).wait()
        pltpu.make_async_copy(dst_hbm.at[pl.ds(0, CH)], dstv[b],
                              semi[b]).wait()

    def issue_gather(b):
        pltpu.async_copy(y_hbm.at[srcv[b]], rows[b], semg[b])

    def wait_gather(b):
        pltpu.make_async_copy(y_hbm.at[srcv[b]], rows[b], semg[b]).wait()

    issue_idx(0, 0)
    issue_idx(1, 1)
    issue_idx(2, 2)
    wait_idx(0)
    issue_gather(0)
    wait_idx(1)
    issue_gather(1)

    def body(jj, carry):
        for b in (0, 1, 2):
            j = 3 * jj + b
            wait_gather(b)

            @pl.when(j + 2 < nf)
            def _():
                wait_idx((b + 2) % 3)
                issue_gather((b + 2) % 3)

            pltpu.sync_copy(rows[b], accum.at[dstv[b]], add=True)

            @pl.when(j + 3 < nf)
            def _():
                issue_idx(j + 3, b)
        return carry

    lax.fori_loop(0, nf // 3, body, 0)


def _tail_chunk(y_hbm, src_hbm, dst_hbm, accum, srcv_t, dstv_t, rows_t, sem,
                src_off, dst_off):
    pltpu.sync_copy(src_hbm.at[pl.ds(src_off, srcv_t.shape[0])], srcv_t)
    pltpu.sync_copy(dst_hbm.at[pl.ds(dst_off, dstv_t.shape[0])], dstv_t)
    pltpu.async_copy(y_hbm.at[srcv_t], rows_t, sem).wait()
    pltpu.sync_copy(rows_t, accum.at[dstv_t], add=True)


def _sc_scratch(W, tail):
    return [
        pltpu.VMEM((CH,), jnp.int32),
        pltpu.VMEM((CH,), jnp.int32),
        pltpu.VMEM((CH,), jnp.int32),
        pltpu.VMEM((CH,), jnp.int32),
        pltpu.VMEM((CH,), jnp.int32),
        pltpu.VMEM((CH,), jnp.int32),
        pltpu.VMEM((CH, W), jnp.float32),
        pltpu.VMEM((CH, W), jnp.float32),
        pltpu.VMEM((CH, W), jnp.float32),
        pltpu.VMEM((tail,), jnp.int32),
        pltpu.VMEM((tail,), jnp.int32),
        pltpu.VMEM((tail, W), jnp.float32),
        pltpu.VMEM((RPT_LAST, W), jnp.float32),
        pltpu.VMEM_SHARED((N, W), jnp.float32),
        pltpu.SemaphoreType.DMA,
        pltpu.SemaphoreType.DMA,
        pltpu.SemaphoreType.DMA,
        pltpu.SemaphoreType.DMA,
        pltpu.SemaphoreType.DMA,
        pltpu.SemaphoreType.DMA,
    ]


_MESH = plsc.VectorSubcoreMesh(core_axis_name="c", subcore_axis_name="s")
_SC_PARAMS = pltpu.CompilerParams(use_tc_tiling_on_sc=False)


@functools.partial(
    pl.kernel,
    out_type=jax.ShapeDtypeStruct((2 * N, D // 2), jnp.float32),
    mesh=_MESH,
    scratch_types=_sc_scratch(D // 2, TAIL1),
    compiler_params=_SC_PARAMS,
)
def _sc_agg1(y_hbm, src_hbm, dst_hbm, zeros_hbm, out_hbm,
             sv0, sv1, sv2, dv0, dv1, dv2, r0, r1, r2,
             srcv_t, dstv_t, rows_t, stage, accum,
             si0, si1, si2, sg0, sg1, sg2):
    c = lax.axis_index("c")
    s = lax.axis_index("s")
    _zero_accum(zeros_hbm, accum, stage, s)
    plsc.subcore_barrier()
    src_base = pl.multiple_of(c * E + s * EPT1, 8)
    dst_base = pl.multiple_of(s * EPT1, 8)
    _edge_loop(y_hbm, src_hbm, dst_hbm, accum,
               (sv0, sv1, sv2), (dv0, dv1, dv2), (r0, r1, r2),
               (si0, si1, si2), (sg0, sg1, sg2), src_base, dst_base, NF1)
    _tail_chunk(y_hbm, src_hbm, dst_hbm, accum, srcv_t, dstv_t, rows_t, sg0,
                pl.multiple_of(src_base + NF1 * CH, 8),
                pl.multiple_of(dst_base + NF1 * CH, 8))
    plsc.subcore_barrier()
    _readout(accum, out_hbm, stage, s, c)


@functools.partial(
    pl.kernel,
    out_type=jax.ShapeDtypeStruct((2 * N, H), jnp.float32),
    mesh=_MESH,
    scratch_types=_sc_scratch(H, TAIL2),
    compiler_params=_SC_PARAMS,
)
def _sc_agg2(y_hbm, src_hbm, dst_hbm, zeros_hbm, out_hbm,
             sv0, sv1, sv2, dv0, dv1, dv2, r0, r1, r2,
             srcv_t, dstv_t, rows_t, stage, accum,
             si0, si1, si2, sg0, sg1, sg2):
    c = lax.axis_index("c")
    s = lax.axis_index("s")
    _zero_accum(zeros_hbm, accum, stage, s)
    plsc.subcore_barrier()
    base = pl.multiple_of((c * NS + s) * EPT2, 8)
    _edge_loop(y_hbm, src_hbm, dst_hbm, accum,
               (sv0, sv1, sv2), (dv0, dv1, dv2), (r0, r1, r2),
               (si0, si1, si2), (sg0, sg1, sg2), base, base, NF2)
    _tail_chunk(y_hbm, src_hbm, dst_hbm, accum, srcv_t, dstv_t, rows_t, sg0,
                pl.multiple_of(base + NF2 * CH, 8),
                pl.multiple_of(base + NF2 * CH, 8))
    plsc.subcore_barrier()
    _readout(accum, out_hbm, stage, s, c)


def _tc_conv1(x, aggL, aggR, W1a, b1a, W1b, b1b):
    def body(x_ref, al_ref, ar_ref, wa_ref, ba_ref, wb_ref, bb_ref, o_ref):
        agg = jnp.concatenate([al_ref[...], ar_ref[...]], axis=1)
        h = x_ref[...] + agg
        u = jnp.maximum(jnp.dot(h, wa_ref[...],
                                preferred_element_type=jnp.float32)
                        + ba_ref[...], 0.0)
        v = jnp.dot(u, wb_ref[...], preferred_element_type=jnp.float32)
        o_ref[...] = jnp.maximum(v + bb_ref[...], 0.0)
    return pl.pallas_call(
        body, out_shape=jax.ShapeDtypeStruct((N, H), jnp.float32))(
            x, aggL, aggR, W1a, b1a.reshape(1, H), W1b, b1b.reshape(1, H))


def _tc_conv2(h1, q0, q1, W2a, b2a, W2b, b2b, Wf, bf):
    def body(h_ref, q0_ref, q1_ref, wa_ref, ba_ref, wb_ref, bb_ref,
             wf_ref, bf_ref, o_ref):
        g = h_ref[...] + (q0_ref[...] + q1_ref[...])
        u = jnp.maximum(jnp.dot(g, wa_ref[...],
                                preferred_element_type=jnp.float32)
                        + ba_ref[...], 0.0)
        v = jnp.dot(u, wb_ref[...], preferred_element_type=jnp.float32)
        h2 = jnp.maximum(v + bb_ref[...], 0.0)
        o_ref[...] = jnp.dot(h2, wf_ref[...],
                             preferred_element_type=jnp.float32) + bf_ref[...]
    return pl.pallas_call(
        body, out_shape=jax.ShapeDtypeStruct((N, 1), jnp.float32))(
            h1, q0, q1, W2a, b2a.reshape(1, H), W2b, b2b.reshape(1, H),
            Wf, bf.reshape(1, 1))


def kernel(x, edge_index, W1a, b1a, W1b, b1b, W2a, b2a, W2b, b2b, Wf, bf):
    src = edge_index[0]
    dst = edge_index[1]
    x2 = x.reshape(2 * N, D // 2)
    src_eo = jnp.concatenate([src * 2, src * 2 + 1])
    zeros_d = jnp.zeros((N, D // 2), jnp.float32)
    zeros_h = jnp.zeros((N, H), jnp.float32)

    agg1 = _sc_agg1(x2, src_eo, dst, zeros_d)
    h1 = _tc_conv1(x, agg1[:N], agg1[N:], W1a, b1a, W1b, b1b)
    parts2 = _sc_agg2(h1, src, dst, zeros_h)
    return _tc_conv2(h1, parts2[:N], parts2[N:], W2a, b2a, W2b, b2b, Wf, bf)


# speedup vs baseline: 13.8819x; 1.2573x over previous
import functools

import jax
import jax.numpy as jnp
from jax import lax
from jax.experimental import pallas as pl
from jax.experimental.pallas import tpu as pltpu
from jax.experimental.pallas import tpu_sc as plsc

N = 10000
E = 320000
D = 128
H = 32

NC = 2
NS = 16
CH = 128
RPT = 624
RPT_LAST = N - 15 * RPT

EPT1 = E // NS
NF1 = EPT1 // CH
TAIL1 = EPT1 - NF1 * CH

EPT2 = E // (NC * NS)
NF2 = EPT2 // CH
TAIL2 = EPT2 - NF2 * CH


def _zero_accum(zeros_hbm, accum, s):
    row0 = pl.multiple_of(s * RPT, 8)

    @pl.when(s < NS - 1)
    def _():
        pltpu.sync_copy(zeros_hbm.at[pl.ds(row0, RPT)],
                        accum.at[pl.ds(row0, RPT)])

    @pl.when(s == NS - 1)
    def _():
        pltpu.sync_copy(zeros_hbm.at[pl.ds(row0, RPT_LAST)],
                        accum.at[pl.ds(row0, RPT_LAST)])


def _readout(accum, out_hbm, s, slab):
    row0 = pl.multiple_of(s * RPT, 8)
    out0 = pl.multiple_of(slab * N + s * RPT, 8)

    @pl.when(s < NS - 1)
    def _():
        pltpu.sync_copy(accum.at[pl.ds(row0, RPT)],
                        out_hbm.at[pl.ds(out0, RPT)])

    @pl.when(s == NS - 1)
    def _():
        pltpu.sync_copy(accum.at[pl.ds(row0, RPT_LAST)],
                        out_hbm.at[pl.ds(out0, RPT_LAST)])


def _edge_loop(y_hbm, src2d, dst2d, accum, rows, semg, nf):
    def issue_gather(j, b):
        pltpu.async_copy(y_hbm.at[src2d.at[j]], rows[b], semg[b])

    def wait_gather(j, b):
        pltpu.make_async_copy(y_hbm.at[src2d.at[j]], rows[b], semg[b]).wait()

    issue_gather(0, 0)
    issue_gather(1, 1)

    def body(jj, carry):
        for b in (0, 1, 2):
            j = 3 * jj + b
            wait_gather(j, b)

            @pl.when(j + 2 < nf)
            def _():
                issue_gather(j + 2, (b + 2) % 3)

            pltpu.sync_copy(rows[b], accum.at[dst2d.at[j]], add=True)
        return carry

    lax.fori_loop(0, nf // 3, body, 0)


def _tail_chunk(y_hbm, src_hbm, dst_hbm, accum, srcv_t, dstv_t, rows_t, sem,
                src_off, dst_off):
    pltpu.sync_copy(src_hbm.at[pl.ds(src_off, srcv_t.shape[0])], srcv_t)
    pltpu.sync_copy(dst_hbm.at[pl.ds(dst_off, dstv_t.shape[0])], dstv_t)
    pltpu.async_copy(y_hbm.at[srcv_t], rows_t, sem).wait()
    pltpu.sync_copy(rows_t, accum.at[dstv_t], add=True)


def _sc_scratch(W, nf, tail):
    return [
        pltpu.VMEM((nf, CH), jnp.int32),
        pltpu.VMEM((nf, CH), jnp.int32),
        pltpu.VMEM((CH, W), jnp.float32),
        pltpu.VMEM((CH, W), jnp.float32),
        pltpu.VMEM((CH, W), jnp.float32),
        pltpu.VMEM((tail,), jnp.int32),
        pltpu.VMEM((tail,), jnp.int32),
        pltpu.VMEM((tail, W), jnp.float32),
        pltpu.VMEM_SHARED((N, W), jnp.float32),
        pltpu.SemaphoreType.DMA,
        pltpu.SemaphoreType.DMA,
        pltpu.SemaphoreType.DMA,
        pltpu.SemaphoreType.DMA,
    ]


_MESH = plsc.VectorSubcoreMesh(core_axis_name="c", subcore_axis_name="s")
_SC_PARAMS = pltpu.CompilerParams(use_tc_tiling_on_sc=False)


@functools.partial(
    pl.kernel,
    out_type=jax.ShapeDtypeStruct((2 * N, D // 2), jnp.float32),
    mesh=_MESH,
    scratch_types=_sc_scratch(D // 2, NF1, TAIL1),
    compiler_params=_SC_PARAMS,
)
def _sc_agg1(y_hbm, src3d_hbm, dst3d_hbm, src_hbm, dst_hbm, zeros_hbm,
             out_hbm, src2d, dst2d, r0, r1, r2, srcv_t, dstv_t, rows_t,
             accum, semi, sg0, sg1, sg2):
    c = lax.axis_index("c")
    s = lax.axis_index("s")
    pltpu.async_copy(src3d_hbm.at[c * NS + s], src2d, semi)
    pltpu.async_copy(dst3d_hbm.at[s], dst2d, semi)
    _zero_accum(zeros_hbm, accum, s)
    plsc.subcore_barrier()
    pltpu.make_async_copy(src3d_hbm.at[0], src2d, semi).wait()
    pltpu.make_async_copy(dst3d_hbm.at[0], dst2d, semi).wait()
    _edge_loop(y_hbm, src2d, dst2d, accum, (r0, r1, r2),
               (sg0, sg1, sg2), NF1)
    _tail_chunk(y_hbm, src_hbm, dst_hbm, accum, srcv_t, dstv_t, rows_t, sg0,
                pl.multiple_of(c * E + s * EPT1 + NF1 * CH, 8),
                pl.multiple_of(s * EPT1 + NF1 * CH, 8))
    plsc.subcore_barrier()
    _readout(accum, out_hbm, s, c)


@functools.partial(
    pl.kernel,
    out_type=jax.ShapeDtypeStruct((2 * N, H), jnp.float32),
    mesh=_MESH,
    scratch_types=_sc_scratch(H, NF2, TAIL2),
    compiler_params=_SC_PARAMS,
)
def _sc_agg2(y_hbm, src3d_hbm, dst3d_hbm, src_hbm, dst_hbm, zeros_hbm,
             out_hbm, src2d, dst2d, r0, r1, r2, srcv_t, dstv_t, rows_t,
             accum, semi, sg0, sg1, sg2):
    c = lax.axis_index("c")
    s = lax.axis_index("s")
    w = c * NS + s
    pltpu.async_copy(src3d_hbm.at[w], src2d, semi)
    pltpu.async_copy(dst3d_hbm.at[w], dst2d, semi)
    _zero_accum(zeros_hbm, accum, s)
    plsc.subcore_barrier()
    pltpu.make_async_copy(src3d_hbm.at[0], src2d, semi).wait()
    pltpu.make_async_copy(dst3d_hbm.at[0], dst2d, semi).wait()
    _edge_loop(y_hbm, src2d, dst2d, accum, (r0, r1, r2),
               (sg0, sg1, sg2), NF2)
    _tail_chunk(y_hbm, src_hbm, dst_hbm, accum, srcv_t, dstv_t, rows_t, sg0,
                pl.multiple_of(w * EPT2 + NF2 * CH, 8),
                pl.multiple_of(w * EPT2 + NF2 * CH, 8))
    plsc.subcore_barrier()
    _readout(accum, out_hbm, s, c)


def _tc_conv1(x, aggL, aggR, W1a, b1a, W1b, b1b):
    def body(x_ref, al_ref, ar_ref, wa_ref, ba_ref, wb_ref, bb_ref, o_ref):
        agg = jnp.concatenate([al_ref[...], ar_ref[...]], axis=1)
        h = x_ref[...] + agg
        u = jnp.maximum(jnp.dot(h, wa_ref[...],
                                preferred_element_type=jnp.float32)
                        + ba_ref[...], 0.0)
        v = jnp.dot(u, wb_ref[...], preferred_element_type=jnp.float32)
        o_ref[...] = jnp.maximum(v + bb_ref[...], 0.0)
    return pl.pallas_call(
        body, out_shape=jax.ShapeDtypeStruct((N, H), jnp.float32))(
            x, aggL, aggR, W1a, b1a.reshape(1, H), W1b, b1b.reshape(1, H))


def _tc_conv2(h1, q0, q1, W2a, b2a, W2b, b2b, Wf, bf):
    def body(h_ref, q0_ref, q1_ref, wa_ref, ba_ref, wb_ref, bb_ref,
             wf_ref, bf_ref, o_ref):
        g = h_ref[...] + (q0_ref[...] + q1_ref[...])
        u = jnp.maximum(jnp.dot(g, wa_ref[...],
                                preferred_element_type=jnp.float32)
                        + ba_ref[...], 0.0)
        v = jnp.dot(u, wb_ref[...], preferred_element_type=jnp.float32)
        h2 = jnp.maximum(v + bb_ref[...], 0.0)
        o_ref[...] = jnp.dot(h2, wf_ref[...],
                             preferred_element_type=jnp.float32) + bf_ref[...]
    return pl.pallas_call(
        body, out_shape=jax.ShapeDtypeStruct((N, 1), jnp.float32))(
            h1, q0, q1, W2a, b2a.reshape(1, H), W2b, b2b.reshape(1, H),
            Wf, bf.reshape(1, 1))


def kernel(x, edge_index, W1a, b1a, W1b, b1b, W2a, b2a, W2b, b2b, Wf, bf):
    src = edge_index[0]
    dst = edge_index[1]
    x2 = x.reshape(2 * N, D // 2)
    src_eo = jnp.concatenate([src * 2, src * 2 + 1])
    zeros_d = jnp.zeros((N, D // 2), jnp.float32)
    zeros_h = jnp.zeros((N, H), jnp.float32)

    src3d_1 = src_eo.reshape(NC * NS, EPT1)[:, :NF1 * CH].reshape(
        NC * NS, NF1, CH)
    dst3d_1 = dst.reshape(NS, EPT1)[:, :NF1 * CH].reshape(NS, NF1, CH)
    src3d_2 = src.reshape(NC * NS, EPT2)[:, :NF2 * CH].reshape(
        NC * NS, NF2, CH)
    dst3d_2 = dst.reshape(NC * NS, EPT2)[:, :NF2 * CH].reshape(
        NC * NS, NF2, CH)

    agg1 = _sc_agg1(x2, src3d_1, dst3d_1, src_eo, dst, zeros_d)
    h1 = _tc_conv1(x, agg1[:N], agg1[N:], W1a, b1a, W1b, b1b)
    parts2 = _sc_agg2(h1, src3d_2, dst3d_2, src, dst, zeros_h)
    return _tc_conv2(h1, parts2[:N], parts2[N:], W2a, b2a, W2b, b2b, Wf, bf)
